# baseline (speedup 1.0000x reference)
import jax
import jax.numpy as jnp
from jax.experimental import pallas as pl
from jax.experimental.pallas import tpu as pltpu

_N_HEADS = 32
_MAX_CTX = 8192
_HDIM = 128
_QLEN = 16
_WIN = 8
_NCHUNK = 8
_ROWS = _N_HEADS * _MAX_CTX
_CHUNK = _ROWS // _NCHUNK


def _copy_body(kc_ref, vc_ref, ko_ref, vo_ref, sem):
    copies = []
    for c in range(_NCHUNK):
        sl = pl.ds(c * _CHUNK, _CHUNK)
        copies.append(pltpu.make_async_copy(kc_ref.at[sl], ko_ref.at[sl], sem))
        copies.append(pltpu.make_async_copy(vc_ref.at[sl], vo_ref.at[sl], sem))
    for cp in copies:
        cp.start()
    for cp in copies:
        cp.wait()


def _dma_copy(kc2, vc2):
    any_spec = pl.BlockSpec(memory_space=pltpu.HBM)
    return pl.pallas_call(
        _copy_body,
        in_specs=[any_spec, any_spec],
        out_specs=[any_spec, any_spec],
        out_shape=[
            jax.ShapeDtypeStruct(kc2.shape, kc2.dtype),
            jax.ShapeDtypeStruct(vc2.shape, vc2.dtype),
        ],
        scratch_shapes=[pltpu.SemaphoreType.DMA],
    )(kc2, vc2)


def _scatter_kernel(pos_ref, kc_ref, vc_ref, k_ref, v_ref, ko_ref, vo_ref):
    i = pl.program_id(0)
    w = pos_ref[i] // _WIN
    ko_ref[...] = kc_ref[...]
    vo_ref[...] = vc_ref[...]
    for j in range(_QLEN):
        pj = pos_ref[j]

        @pl.when(pj // _WIN == w)
        def _():
            r = pj % _WIN
            ko_ref[0, :, r, :] = k_ref[0, :, j, :]
            vo_ref[0, :, r, :] = v_ref[0, :, j, :]


def kernel(k_cache, v_cache, pos_ids, k, v):
    kc2 = k_cache.reshape(_ROWS, _HDIM)
    vc2 = v_cache.reshape(_ROWS, _HDIM)
    kcp, vcp = _dma_copy(kc2, vc2)
    kcp = kcp.reshape(k_cache.shape)
    vcp = vcp.reshape(v_cache.shape)

    win_spec = pl.BlockSpec((1, _N_HEADS, _WIN, _HDIM),
                            lambda i, pos_ref: (0, 0, pos_ref[i] // _WIN, 0))
    kv_spec = pl.BlockSpec((1, _N_HEADS, _QLEN, _HDIM),
                           lambda i, pos_ref: (0, 0, 0, 0))
    ko, vo = pl.pallas_call(
        _scatter_kernel,
        grid_spec=pltpu.PrefetchScalarGridSpec(
            num_scalar_prefetch=1,
            grid=(_QLEN,),
            in_specs=[win_spec, win_spec, kv_spec, kv_spec],
            out_specs=[win_spec, win_spec],
        ),
        out_shape=[
            jax.ShapeDtypeStruct(k_cache.shape, k_cache.dtype),
            jax.ShapeDtypeStruct(v_cache.shape, v_cache.dtype),
        ],
        input_output_aliases={1: 0, 2: 1},
    )(pos_ids.astype(jnp.int32), kcp, vcp, k, v)
    return (ko, vo)


# speedup vs baseline: 43.7400x; 43.7400x over previous
import functools

import jax
import jax.numpy as jnp
from jax import lax
from jax.experimental import pallas as pl
from jax.experimental.pallas import tpu as pltpu
from jax.experimental.pallas import tpu_sc as plsc

_N_HEADS = 32
_MAX_CTX = 8192
_HDIM = 128
_QLEN = 16
_BLK = 8192
_NBLK = _MAX_CTX // _BLK


def _copy_kernel(kc_ref, vc_ref, ko_ref, vo_ref):
    ko_ref[...] = kc_ref[...]
    vo_ref[...] = vc_ref[...]


def _tc_copy(k_cache, v_cache):
    spec = pl.BlockSpec((1, 1, _BLK, _HDIM), lambda h, j: (0, h, j, 0))
    return pl.pallas_call(
        _copy_kernel,
        grid=(_N_HEADS, _NBLK),
        in_specs=[spec, spec],
        out_specs=[spec, spec],
        out_shape=[
            jax.ShapeDtypeStruct(k_cache.shape, k_cache.dtype),
            jax.ShapeDtypeStruct(v_cache.shape, v_cache.dtype),
        ],
        compiler_params=pltpu.CompilerParams(
            dimension_semantics=("parallel", "parallel"),
        ),
    )(k_cache, v_cache)


def _sc_body(ko_hbm, vo_hbm, pos_hbm, k_hbm, v_hbm,
             idx_v, src_v, krows_v, vrows_v, sem_k, sem_v):
    wid = lax.axis_index("s") * 2 + lax.axis_index("c")
    row0 = wid * _MAX_CTX
    pltpu.sync_copy(pos_hbm, idx_v)
    pvec = idx_v[...]
    m = lax.iota(jnp.int32, _QLEN)
    for j in range(_QLEN):
        bj = pvec.at[jnp.full((_QLEN,), j, jnp.int32)].get(
            mode="promise_in_bounds")
        m = jnp.where(pvec == bj, jnp.maximum(m, j), m)
    src_v[...] = m + wid * _QLEN
    idx_v[...] = pvec + row0
    gk = pltpu.make_async_copy(k_hbm.at[src_v], krows_v, sem_k)
    gv = pltpu.make_async_copy(v_hbm.at[src_v], vrows_v, sem_v)
    gk.start()
    gv.start()
    gk.wait()
    gv.wait()
    sk = pltpu.make_async_copy(krows_v, ko_hbm.at[idx_v], sem_k)
    sv = pltpu.make_async_copy(vrows_v, vo_hbm.at[idx_v], sem_v)
    sk.start()
    sv.start()
    sk.wait()
    sv.wait()


_sc_scatter = functools.partial(
    pl.kernel,
    mesh=plsc.VectorSubcoreMesh(core_axis_name="c", subcore_axis_name="s"),
    scratch_types=[
        pltpu.VMEM((_QLEN,), jnp.int32),
        pltpu.VMEM((_QLEN,), jnp.int32),
        pltpu.VMEM((_QLEN, _HDIM), jnp.float32),
        pltpu.VMEM((_QLEN, _HDIM), jnp.float32),
        pltpu.SemaphoreType.DMA,
        pltpu.SemaphoreType.DMA,
    ],
)(_sc_body)


def kernel(k_cache, v_cache, pos_ids, k, v):
    ko, vo = _tc_copy(k_cache, v_cache)
    ko_ref = jax.new_ref(ko.reshape(_N_HEADS * _MAX_CTX, _HDIM))
    vo_ref = jax.new_ref(vo.reshape(_N_HEADS * _MAX_CTX, _HDIM))
    _sc_scatter(ko_ref, vo_ref,
                pos_ids.astype(jnp.int32),
                k.reshape(_N_HEADS * _QLEN, _HDIM),
                v.reshape(_N_HEADS * _QLEN, _HDIM))
    return (ko_ref[...].reshape(k_cache.shape),
            vo_ref[...].reshape(v_cache.shape))
